# baseline (speedup 1.0000x reference)
import functools

import jax
import jax.numpy as jnp
from jax import lax
from jax.experimental import pallas as pl
from jax.experimental.pallas import tpu as pltpu
from jax.experimental.pallas import tpu_sc as plsc

N = 10000
E = 320000
D = 128
G = 16

NC = 2
NS = 16
NW = NC * NS
NP = 10240
RPT = NP // NS
EPW = 10240
EP = NW * EPW
CH = 64
NCHUNK = EPW // CH
BN = 512
NB = NP // BN

f32 = jnp.float32
i32 = jnp.int32

_sc_mesh = plsc.VectorSubcoreMesh(core_axis_name="c", subcore_axis_name="s")


@functools.partial(
    pl.kernel,
    mesh=_sc_mesh,
    out_type=jax.ShapeDtypeStruct((NC, NP, 16), f32),
    scratch_types=[
        pltpu.VMEM((NCHUNK, 2, CH), i32),
        pltpu.VMEM((CH, 16), f32),
        pltpu.VMEM_SHARED((NP, 16), f32),
        pltpu.SemaphoreType.DMA,
    ],
)
def _deg_kernel(ei_hbm, out_hbm, idx_v, buf_v, acc_sh, sem):
    c = lax.axis_index("c")
    s = lax.axis_index("s")
    w = c * NS + s
    row0 = s * RPT

    pltpu.sync_copy(ei_hbm.at[pl.ds(w * NCHUNK, NCHUNK)], idx_v)

    def _fill(val):
        def body(j, _):
            buf_v[j, :] = jnp.full((16,), val, f32)
            return 0
        lax.fori_loop(0, CH, body, 0)

    _fill(0.0)

    def clr(k, _):
        pltpu.sync_copy(buf_v, acc_sh.at[pl.ds(row0 + k * CH, CH)])
        return 0
    lax.fori_loop(0, RPT // CH, clr, 0)
    plsc.subcore_barrier()

    _fill(1.0)

    def step(g, _):
        ds = [pltpu.async_copy(buf_v, acc_sh.at[idx_v.at[8 * g + u, 1]],
                               sem, add=True) for u in range(8)]
        for d in ds:
            d.wait()
        return 0
    lax.fori_loop(0, NCHUNK // 8, step, 0)
    plsc.subcore_barrier()

    def out(k, _):
        r = row0 + k * CH
        pltpu.sync_copy(acc_sh.at[pl.ds(r, CH)], buf_v)
        pltpu.sync_copy(buf_v, out_hbm.at[c, pl.ds(r, CH)])
        return 0
    lax.fori_loop(0, RPT // CH, out, 0)


PCH = 40
TOTC = EP // CH
C_SLOW = 80
C_FAST = 2 * NCHUNK - C_SLOW


@functools.partial(
    pl.kernel,
    mesh=_sc_mesh,
    out_type=jax.ShapeDtypeStruct((NC, NP, D), f32),
    scratch_types=[
        pltpu.VMEM((PCH, 2, CH), i32),
        pltpu.VMEM((CH, D), f32),
        pltpu.VMEM((CH, D), f32),
        pltpu.VMEM((CH, D), f32),
        pltpu.VMEM_SHARED((NP, D), f32),
        pltpu.SemaphoreType.DMA,
        pltpu.SemaphoreType.DMA,
        pltpu.SemaphoreType.DMA,
        pltpu.SemaphoreType.DMA,
        pltpu.SemaphoreType.DMA,
        pltpu.SemaphoreType.DMA,
    ],
)
def _edge_kernel(y_hbm, ei_hbm, out_hbm, idx_v, r0_v, r1_v, r2_v,
                 acc_sh, g0, g1, g2, s0, s1, s2):
    c = lax.axis_index("c")
    s = lax.axis_index("s")
    w = c * NS + s
    row0 = s * RPT
    rows = (r0_v, r1_v, r2_v)
    gsem = (g0, g1, g2)
    ssem = (s0, s1, s2)

    nch = jnp.where(c == 0, C_SLOW, C_FAST)
    base = jnp.where(c == 0, s * C_SLOW, NS * C_SLOW + s * C_FAST)

    def zrow(i, _):
        def zb(j, _):
            r1_v[i, pl.ds(j * 16, 16)] = jnp.zeros((16,), f32)
            return 0
        lax.fori_loop(0, D // 16, zb, 0)
        return 0
    lax.fori_loop(0, CH, zrow, 0)

    def clr(k, _):
        pltpu.sync_copy(r1_v, acc_sh.at[pl.ds(row0 + k * CH, CH)])
        return 0
    lax.fori_loop(0, RPT // CH, clr, 0)
    plsc.subcore_barrier()

    def step(g, _):
        @pl.when(g % (PCH // 8) == 0)
        def _():
            pltpu.sync_copy(ei_hbm.at[pl.ds(base + g * 8, PCH)], idx_v)

        k0 = (g % (PCH // 8)) * 8
        dg = [None] * 8
        ds = [None] * 8
        dg[0] = pltpu.async_copy(y_hbm.at[idx_v.at[k0, 0]],
                                 rows[0], gsem[0])
        dg[1] = pltpu.async_copy(y_hbm.at[idx_v.at[k0 + 1, 0]],
                                 rows[1], gsem[1])
        for u in range(8):
            b = u % 3
            if u >= 1 and u + 2 < 8:
                ds[u - 1].wait()
            if u + 2 < 8:
                dg[u + 2] = pltpu.async_copy(
                    y_hbm.at[idx_v.at[k0 + u + 2, 0]],
                    rows[(u + 2) % 3], gsem[(u + 2) % 3])
            dg[u].wait()
            ds[u] = pltpu.async_copy(
                rows[b], acc_sh.at[idx_v.at[k0 + u, 1]], ssem[b],
                add=True)
        for u in range(5, 8):
            ds[u].wait()
        return 0
    lax.fori_loop(0, nch // 8, step, 0)

    plsc.subcore_barrier()

    def out(p, _):
        r = row0 + 2 * p * CH
        pltpu.sync_copy(acc_sh.at[pl.ds(r, CH)], r0_v)
        d0 = pltpu.async_copy(r0_v, out_hbm.at[c, pl.ds(r, CH)], g0)
        pltpu.sync_copy(acc_sh.at[pl.ds(r + CH, CH)], r1_v)
        d1 = pltpu.async_copy(r1_v, out_hbm.at[c, pl.ds(r + CH, CH)], g1)
        d0.wait()
        d1.wait()
        return 0
    lax.fori_loop(0, RPT // CH // 2, out, 0)


def _k1_body(x_ref, w_ref, dp_ref, y_ref, dis_ref):
    i = pl.program_id(0)
    dp = dp_ref[...]
    dis = lax.rsqrt(dp[0] + dp[1] + 1.0)
    dis_ref[...] = dis
    y = dis[:, :1] * jnp.dot(x_ref[...], w_ref[...], preferred_element_type=f32)
    rows = i * BN + lax.broadcasted_iota(i32, (BN, 1), 0)
    y_ref[...] = jnp.where(rows < N, y, 0.0)


def _k1(xp, w1, degp):
    return pl.pallas_call(
        _k1_body,
        grid=(NB,),
        in_specs=[
            pl.BlockSpec((BN, D), lambda i: (i, 0)),
            pl.BlockSpec((D, D), lambda i: (0, 0)),
            pl.BlockSpec((NC, BN, 16), lambda i: (0, i, 0)),
        ],
        out_specs=[
            pl.BlockSpec((BN, D), lambda i: (i, 0)),
            pl.BlockSpec((BN, 16), lambda i: (i, 0)),
        ],
        out_shape=[
            jax.ShapeDtypeStruct((NP, D), f32),
            jax.ShapeDtypeStruct((NP, 16), f32),
        ],
    )(xp, w1, degp)


def _k23_body(sp_ref, y_ref, dis_ref, b_ref, w_ref, yn_ref):
    i = pl.program_id(0)
    sp = sp_ref[...]
    dis = dis_ref[...][:, :1]
    h = jnp.maximum(dis * (sp[0] + sp[1] + y_ref[...]) + b_ref[...], 0.0)
    yn = dis * jnp.dot(h, w_ref[...], preferred_element_type=f32)
    rows = i * BN + lax.broadcasted_iota(i32, (BN, 1), 0)
    yn_ref[...] = jnp.where(rows < N, yn, 0.0)


def _k23(sp, y, dis, b2d, w):
    return pl.pallas_call(
        _k23_body,
        grid=(NB,),
        in_specs=[
            pl.BlockSpec((NC, BN, D), lambda i: (0, i, 0)),
            pl.BlockSpec((BN, D), lambda i: (i, 0)),
            pl.BlockSpec((BN, 16), lambda i: (i, 0)),
            pl.BlockSpec((1, D), lambda i: (0, 0)),
            pl.BlockSpec((D, D), lambda i: (0, 0)),
        ],
        out_specs=pl.BlockSpec((BN, D), lambda i: (i, 0)),
        out_shape=jax.ShapeDtypeStruct((NP, D), f32),
    )(sp, y, dis, b2d, w)


def _k4_body(sp_ref, y_ref, dis_ref, b_ref, bat_ref, fw1_ref, fb1_ref,
             fw2_ref, fb2_ref, out_ref, acc_ref, cnt_ref):
    i = pl.program_id(0)

    @pl.when(i == 0)
    def _():
        acc_ref[...] = jnp.zeros((G, D), f32)
        cnt_ref[...] = jnp.zeros((G, 128), f32)

    sp = sp_ref[...]
    dis = dis_ref[...][:, :1]
    h = jnp.maximum(dis * (sp[0] + sp[1] + y_ref[...]) + b_ref[...], 0.0)
    onehot = (bat_ref[...][:, :1] == lax.broadcasted_iota(i32, (1, G), 1)
              ).astype(f32)
    acc_ref[...] += lax.dot_general(onehot, h, (((0,), (0,)), ((), ())),
                                    preferred_element_type=f32)
    cnt_ref[...] += jnp.broadcast_to(jnp.sum(onehot, axis=0)[:, None],
                                     (G, 128))

    @pl.when(i == NB - 1)
    def _():
        pooled = acc_ref[...] / jnp.maximum(cnt_ref[...], 1.0)
        z = jnp.maximum(jnp.dot(pooled, fw1_ref[...],
                                preferred_element_type=f32) + fb1_ref[...], 0.0)
        out_ref[...] = jnp.dot(z, fw2_ref[...],
                               preferred_element_type=f32) + fb2_ref[...]


def _k4(sp, y, dis, b2d, bat, fw1, fb1_2d, fw2, fb2_2d):
    return pl.pallas_call(
        _k4_body,
        grid=(NB,),
        in_specs=[
            pl.BlockSpec((NC, BN, D), lambda i: (0, i, 0)),
            pl.BlockSpec((BN, D), lambda i: (i, 0)),
            pl.BlockSpec((BN, 16), lambda i: (i, 0)),
            pl.BlockSpec((1, D), lambda i: (0, 0)),
            pl.BlockSpec((BN, 8), lambda i: (i, 0)),
            pl.BlockSpec((D, D), lambda i: (0, 0)),
            pl.BlockSpec((1, D), lambda i: (0, 0)),
            pl.BlockSpec((D, D), lambda i: (0, 0)),
            pl.BlockSpec((1, D), lambda i: (0, 0)),
        ],
        out_specs=pl.BlockSpec((G, D), lambda i: (0, 0)),
        out_shape=jax.ShapeDtypeStruct((G, D), f32),
        scratch_shapes=[pltpu.VMEM((G, D), f32), pltpu.VMEM((G, 128), f32)],
    )(sp, y, dis, b2d, bat, fw1, fb1_2d, fw2, fb2_2d)


def kernel(x, edge_index, batch, W1, b1, W2, b2, W3, b3, fW1, fb1, fW2, fb2):
    pad_e = EP - E
    srcp = jnp.concatenate([edge_index[0], jnp.full((pad_e,), N, i32)])
    dstp = jnp.concatenate([edge_index[1], jnp.full((pad_e,), N, i32)])
    ei_t = jnp.stack([srcp.reshape(TOTC, CH),
                      dstp.reshape(TOTC, CH)], axis=1)
    xp = jnp.zeros((NP, D), f32).at[:N, :].set(x)
    batp = jnp.broadcast_to(
        jnp.concatenate([batch, jnp.full((NP - N,), G, i32)])[:, None], (NP, 8))

    degp = _deg_kernel(ei_t)
    y1, dis = _k1(xp, W1, degp)
    s1 = _edge_kernel(y1, ei_t)
    y2 = _k23(s1, y1, dis, b1.reshape(1, D), W2)
    s2 = _edge_kernel(y2, ei_t)
    y3 = _k23(s2, y2, dis, b2.reshape(1, D), W3)
    s3 = _edge_kernel(y3, ei_t)
    out = _k4(s3, y3, dis, b3.reshape(1, D), batp,
              fW1, fb1.reshape(1, D), fW2, fb2.reshape(1, D))
    return out


# speedup vs baseline: 1.0583x; 1.0583x over previous
import functools

import jax
import jax.numpy as jnp
from jax import lax
from jax.experimental import pallas as pl
from jax.experimental.pallas import tpu as pltpu
from jax.experimental.pallas import tpu_sc as plsc

N = 10000
E = 320000
D = 128
G = 16

NC = 2
NS = 16
NW = NC * NS
NP = 10240
RPT = NP // NS
EPW = 10240
EP = NW * EPW
CH = 64
NCHUNK = EPW // CH
BN = 512
NB = NP // BN

f32 = jnp.float32
i32 = jnp.int32

_sc_mesh = plsc.VectorSubcoreMesh(core_axis_name="c", subcore_axis_name="s")


@functools.partial(
    pl.kernel,
    mesh=_sc_mesh,
    out_type=jax.ShapeDtypeStruct((NC, NP, 16), f32),
    scratch_types=[
        pltpu.VMEM((NCHUNK, 2, CH), i32),
        pltpu.VMEM((CH, 16), f32),
        pltpu.VMEM_SHARED((NP, 16), f32),
        pltpu.SemaphoreType.DMA,
    ],
)
def _deg_kernel(ei_hbm, out_hbm, idx_v, buf_v, acc_sh, sem):
    c = lax.axis_index("c")
    s = lax.axis_index("s")
    w = c * NS + s
    row0 = s * RPT

    pltpu.sync_copy(ei_hbm.at[pl.ds(w * NCHUNK, NCHUNK)], idx_v)

    def _fill(val):
        def body(j, _):
            buf_v[j, :] = jnp.full((16,), val, f32)
            return 0
        lax.fori_loop(0, CH, body, 0)

    _fill(0.0)

    def clr(k, _):
        pltpu.sync_copy(buf_v, acc_sh.at[pl.ds(row0 + k * CH, CH)])
        return 0
    lax.fori_loop(0, RPT // CH, clr, 0)
    plsc.subcore_barrier()

    _fill(1.0)

    def step(g, _):
        ds = [pltpu.async_copy(buf_v, acc_sh.at[idx_v.at[8 * g + u, 1]],
                               sem, add=True) for u in range(8)]
        for d in ds:
            d.wait()
        return 0
    lax.fori_loop(0, NCHUNK // 8, step, 0)
    plsc.subcore_barrier()

    def out(k, _):
        r = row0 + k * CH
        pltpu.sync_copy(acc_sh.at[pl.ds(r, CH)], buf_v)
        pltpu.sync_copy(buf_v, out_hbm.at[c, pl.ds(r, CH)])
        return 0
    lax.fori_loop(0, RPT // CH, out, 0)


PCH = 40
TOTC = EP // CH
C_SLOW = 80
C_FAST = 2 * NCHUNK - C_SLOW


@functools.partial(
    pl.kernel,
    mesh=_sc_mesh,
    out_type=jax.ShapeDtypeStruct((NC, NP, D), f32),
    scratch_types=[
        pltpu.VMEM((PCH, 2, CH), i32),
        pltpu.VMEM((CH, D), f32),
        pltpu.VMEM((CH, D), f32),
        pltpu.VMEM((CH, D), f32),
        pltpu.VMEM_SHARED((NP, D), f32),
        pltpu.SemaphoreType.DMA,
        pltpu.SemaphoreType.DMA,
        pltpu.SemaphoreType.DMA,
        pltpu.SemaphoreType.DMA,
        pltpu.SemaphoreType.DMA,
        pltpu.SemaphoreType.DMA,
    ],
)
def _edge_kernel(y_hbm, ei_hbm, out_hbm, idx_v, r0_v, r1_v, r2_v,
                 acc_sh, g0, g1, g2, s0, s1, s2):
    c = lax.axis_index("c")
    s = lax.axis_index("s")
    w = c * NS + s
    row0 = s * RPT
    rows = (r0_v, r1_v, r2_v)
    gsem = (g0, g1, g2)
    ssem = (s0, s1, s2)

    nch = jnp.where(c == 1, C_SLOW, C_FAST)
    base = jnp.where(c == 1, s * C_SLOW, NS * C_SLOW + s * C_FAST)

    def zrow(i, _):
        def zb(j, _):
            r1_v[i, pl.ds(j * 16, 16)] = jnp.zeros((16,), f32)
            return 0
        lax.fori_loop(0, D // 16, zb, 0)
        return 0
    lax.fori_loop(0, CH, zrow, 0)

    def clr(k, _):
        pltpu.sync_copy(r1_v, acc_sh.at[pl.ds(row0 + k * CH, CH)])
        return 0
    lax.fori_loop(0, RPT // CH, clr, 0)
    plsc.subcore_barrier()

    def step(g, _):
        @pl.when(g % (PCH // 8) == 0)
        def _():
            pltpu.sync_copy(ei_hbm.at[pl.ds(base + g * 8, PCH)], idx_v)

        k0 = (g % (PCH // 8)) * 8
        dg = [None] * 8
        ds = [None] * 8
        dg[0] = pltpu.async_copy(y_hbm.at[idx_v.at[k0, 0]],
                                 rows[0], gsem[0])
        dg[1] = pltpu.async_copy(y_hbm.at[idx_v.at[k0 + 1, 0]],
                                 rows[1], gsem[1])
        for u in range(8):
            b = u % 3
            if u >= 1 and u + 2 < 8:
                ds[u - 1].wait()
            if u + 2 < 8:
                dg[u + 2] = pltpu.async_copy(
                    y_hbm.at[idx_v.at[k0 + u + 2, 0]],
                    rows[(u + 2) % 3], gsem[(u + 2) % 3])
            dg[u].wait()
            ds[u] = pltpu.async_copy(
                rows[b], acc_sh.at[idx_v.at[k0 + u, 1]], ssem[b],
                add=True)
        for u in range(5, 8):
            ds[u].wait()
        return 0
    lax.fori_loop(0, nch // 8, step, 0)

    plsc.subcore_barrier()

    def out(p, _):
        r = row0 + 2 * p * CH
        pltpu.sync_copy(acc_sh.at[pl.ds(r, CH)], r0_v)
        d0 = pltpu.async_copy(r0_v, out_hbm.at[c, pl.ds(r, CH)], g0)
        pltpu.sync_copy(acc_sh.at[pl.ds(r + CH, CH)], r1_v)
        d1 = pltpu.async_copy(r1_v, out_hbm.at[c, pl.ds(r + CH, CH)], g1)
        d0.wait()
        d1.wait()
        return 0
    lax.fori_loop(0, RPT // CH // 2, out, 0)


def _k1_body(x_ref, w_ref, dp_ref, y_ref, dis_ref):
    i = pl.program_id(0)
    dp = dp_ref[...]
    dis = lax.rsqrt(dp[0] + dp[1] + 1.0)
    dis_ref[...] = dis
    y = dis[:, :1] * jnp.dot(x_ref[...], w_ref[...], preferred_element_type=f32)
    rows = i * BN + lax.broadcasted_iota(i32, (BN, 1), 0)
    y_ref[...] = jnp.where(rows < N, y, 0.0)


def _k1(xp, w1, degp):
    return pl.pallas_call(
        _k1_body,
        grid=(NB,),
        in_specs=[
            pl.BlockSpec((BN, D), lambda i: (i, 0)),
            pl.BlockSpec((D, D), lambda i: (0, 0)),
            pl.BlockSpec((NC, BN, 16), lambda i: (0, i, 0)),
        ],
        out_specs=[
            pl.BlockSpec((BN, D), lambda i: (i, 0)),
            pl.BlockSpec((BN, 16), lambda i: (i, 0)),
        ],
        out_shape=[
            jax.ShapeDtypeStruct((NP, D), f32),
            jax.ShapeDtypeStruct((NP, 16), f32),
        ],
    )(xp, w1, degp)


def _k23_body(sp_ref, y_ref, dis_ref, b_ref, w_ref, yn_ref):
    i = pl.program_id(0)
    sp = sp_ref[...]
    dis = dis_ref[...][:, :1]
    h = jnp.maximum(dis * (sp[0] + sp[1] + y_ref[...]) + b_ref[...], 0.0)
    yn = dis * jnp.dot(h, w_ref[...], preferred_element_type=f32)
    rows = i * BN + lax.broadcasted_iota(i32, (BN, 1), 0)
    yn_ref[...] = jnp.where(rows < N, yn, 0.0)


def _k23(sp, y, dis, b2d, w):
    return pl.pallas_call(
        _k23_body,
        grid=(NB,),
        in_specs=[
            pl.BlockSpec((NC, BN, D), lambda i: (0, i, 0)),
            pl.BlockSpec((BN, D), lambda i: (i, 0)),
            pl.BlockSpec((BN, 16), lambda i: (i, 0)),
            pl.BlockSpec((1, D), lambda i: (0, 0)),
            pl.BlockSpec((D, D), lambda i: (0, 0)),
        ],
        out_specs=pl.BlockSpec((BN, D), lambda i: (i, 0)),
        out_shape=jax.ShapeDtypeStruct((NP, D), f32),
    )(sp, y, dis, b2d, w)


def _k4_body(sp_ref, y_ref, dis_ref, b_ref, bat_ref, fw1_ref, fb1_ref,
             fw2_ref, fb2_ref, out_ref, acc_ref, cnt_ref):
    i = pl.program_id(0)

    @pl.when(i == 0)
    def _():
        acc_ref[...] = jnp.zeros((G, D), f32)
        cnt_ref[...] = jnp.zeros((G, 128), f32)

    sp = sp_ref[...]
    dis = dis_ref[...][:, :1]
    h = jnp.maximum(dis * (sp[0] + sp[1] + y_ref[...]) + b_ref[...], 0.0)
    onehot = (bat_ref[...][:, :1] == lax.broadcasted_iota(i32, (1, G), 1)
              ).astype(f32)
    acc_ref[...] += lax.dot_general(onehot, h, (((0,), (0,)), ((), ())),
                                    preferred_element_type=f32)
    cnt_ref[...] += jnp.broadcast_to(jnp.sum(onehot, axis=0)[:, None],
                                     (G, 128))

    @pl.when(i == NB - 1)
    def _():
        pooled = acc_ref[...] / jnp.maximum(cnt_ref[...], 1.0)
        z = jnp.maximum(jnp.dot(pooled, fw1_ref[...],
                                preferred_element_type=f32) + fb1_ref[...], 0.0)
        out_ref[...] = jnp.dot(z, fw2_ref[...],
                               preferred_element_type=f32) + fb2_ref[...]


def _k4(sp, y, dis, b2d, bat, fw1, fb1_2d, fw2, fb2_2d):
    return pl.pallas_call(
        _k4_body,
        grid=(NB,),
        in_specs=[
            pl.BlockSpec((NC, BN, D), lambda i: (0, i, 0)),
            pl.BlockSpec((BN, D), lambda i: (i, 0)),
            pl.BlockSpec((BN, 16), lambda i: (i, 0)),
            pl.BlockSpec((1, D), lambda i: (0, 0)),
            pl.BlockSpec((BN, 8), lambda i: (i, 0)),
            pl.BlockSpec((D, D), lambda i: (0, 0)),
            pl.BlockSpec((1, D), lambda i: (0, 0)),
            pl.BlockSpec((D, D), lambda i: (0, 0)),
            pl.BlockSpec((1, D), lambda i: (0, 0)),
        ],
        out_specs=pl.BlockSpec((G, D), lambda i: (0, 0)),
        out_shape=jax.ShapeDtypeStruct((G, D), f32),
        scratch_shapes=[pltpu.VMEM((G, D), f32), pltpu.VMEM((G, 128), f32)],
    )(sp, y, dis, b2d, bat, fw1, fb1_2d, fw2, fb2_2d)


def kernel(x, edge_index, batch, W1, b1, W2, b2, W3, b3, fW1, fb1, fW2, fb2):
    pad_e = EP - E
    srcp = jnp.concatenate([edge_index[0], jnp.full((pad_e,), N, i32)])
    dstp = jnp.concatenate([edge_index[1], jnp.full((pad_e,), N, i32)])
    ei_t = jnp.stack([srcp.reshape(TOTC, CH),
                      dstp.reshape(TOTC, CH)], axis=1)
    xp = jnp.zeros((NP, D), f32).at[:N, :].set(x)
    batp = jnp.broadcast_to(
        jnp.concatenate([batch, jnp.full((NP - N,), G, i32)])[:, None], (NP, 8))

    degp = _deg_kernel(ei_t)
    y1, dis = _k1(xp, W1, degp)
    s1 = _edge_kernel(y1, ei_t)
    y2 = _k23(s1, y1, dis, b1.reshape(1, D), W2)
    s2 = _edge_kernel(y2, ei_t)
    y3 = _k23(s2, y2, dis, b2.reshape(1, D), W3)
    s3 = _edge_kernel(y3, ei_t)
    out = _k4(s3, y3, dis, b3.reshape(1, D), batp,
              fW1, fb1.reshape(1, D), fW2, fb2.reshape(1, D))
    return out


# speedup vs baseline: 1.0599x; 1.0015x over previous
import functools

import jax
import jax.numpy as jnp
from jax import lax
from jax.experimental import pallas as pl
from jax.experimental.pallas import tpu as pltpu
from jax.experimental.pallas import tpu_sc as plsc

N = 10000
E = 320000
D = 128
G = 16

NC = 2
NS = 16
NW = NC * NS
NP = 10240
RPT = NP // NS
EPW = 10240
EP = NW * EPW
CH = 64
NCHUNK = EPW // CH
BN = 512
NB = NP // BN

f32 = jnp.float32
i32 = jnp.int32

_sc_mesh = plsc.VectorSubcoreMesh(core_axis_name="c", subcore_axis_name="s")


@functools.partial(
    pl.kernel,
    mesh=_sc_mesh,
    out_type=jax.ShapeDtypeStruct((NC, NP, 16), f32),
    scratch_types=[
        pltpu.VMEM((NCHUNK, 2, CH), i32),
        pltpu.VMEM((CH, 16), f32),
        pltpu.VMEM_SHARED((NP, 16), f32),
        pltpu.SemaphoreType.DMA,
    ],
)
def _deg_kernel(ei_hbm, out_hbm, idx_v, buf_v, acc_sh, sem):
    c = lax.axis_index("c")
    s = lax.axis_index("s")
    w = c * NS + s
    row0 = s * RPT

    pltpu.sync_copy(ei_hbm.at[pl.ds(w * NCHUNK, NCHUNK)], idx_v)

    def _fill(val):
        def body(j, _):
            buf_v[j, :] = jnp.full((16,), val, f32)
            return 0
        lax.fori_loop(0, CH, body, 0)

    _fill(0.0)

    def clr(k, _):
        pltpu.sync_copy(buf_v, acc_sh.at[pl.ds(row0 + k * CH, CH)])
        return 0
    lax.fori_loop(0, RPT // CH, clr, 0)
    plsc.subcore_barrier()

    _fill(1.0)

    def step(g, _):
        ds = [pltpu.async_copy(buf_v, acc_sh.at[idx_v.at[8 * g + u, 1]],
                               sem, add=True) for u in range(8)]
        for d in ds:
            d.wait()
        return 0
    lax.fori_loop(0, NCHUNK // 8, step, 0)
    plsc.subcore_barrier()

    def out(k, _):
        r = row0 + k * CH
        pltpu.sync_copy(acc_sh.at[pl.ds(r, CH)], buf_v)
        pltpu.sync_copy(buf_v, out_hbm.at[c, pl.ds(r, CH)])
        return 0
    lax.fori_loop(0, RPT // CH, out, 0)


PCH = 40
TOTC = EP // CH


@functools.partial(
    pl.kernel,
    mesh=_sc_mesh,
    out_type=jax.ShapeDtypeStruct((NC, NP, D), f32),
    scratch_types=[
        pltpu.VMEM((PCH, 2, CH), i32),
        pltpu.VMEM((CH, D), f32),
        pltpu.VMEM((CH, D), f32),
        pltpu.VMEM((CH, D), f32),
        pltpu.VMEM_SHARED((NP, D), f32),
        pltpu.SemaphoreType.DMA,
        pltpu.SemaphoreType.DMA,
        pltpu.SemaphoreType.DMA,
        pltpu.SemaphoreType.DMA,
        pltpu.SemaphoreType.DMA,
        pltpu.SemaphoreType.DMA,
    ],
)
def _edge_kernel(y_hbm, ei_hbm, out_hbm, idx_v, r0_v, r1_v, r2_v,
                 acc_sh, g0, g1, g2, s0, s1, s2):
    c = lax.axis_index("c")
    s = lax.axis_index("s")
    w = c * NS + s
    row0 = s * RPT
    rows = (r0_v, r1_v, r2_v)
    gsem = (g0, g1, g2)
    ssem = (s0, s1, s2)

    nch = NCHUNK
    base = w * NCHUNK

    def zrow(i, _):
        def zb(j, _):
            r1_v[i, pl.ds(j * 16, 16)] = jnp.zeros((16,), f32)
            return 0
        lax.fori_loop(0, D // 16, zb, 0)
        return 0
    lax.fori_loop(0, CH, zrow, 0)

    def clr(k, _):
        pltpu.sync_copy(r1_v, acc_sh.at[pl.ds(row0 + k * CH, CH)])
        return 0
    lax.fori_loop(0, RPT // CH, clr, 0)
    plsc.subcore_barrier()

    def step(g, _):
        @pl.when(g % (PCH // 8) == 0)
        def _():
            pltpu.sync_copy(ei_hbm.at[pl.ds(base + g * 8, PCH)], idx_v)

        k0 = (g % (PCH // 8)) * 8
        dg = [None] * 8
        ds = [None] * 8
        dg[0] = pltpu.async_copy(y_hbm.at[idx_v.at[k0, 0]],
                                 rows[0], gsem[0])
        dg[1] = pltpu.async_copy(y_hbm.at[idx_v.at[k0 + 1, 0]],
                                 rows[1], gsem[1])
        for u in range(8):
            b = u % 3
            if u >= 1 and u + 2 < 8:
                ds[u - 1].wait()
            if u + 2 < 8:
                dg[u + 2] = pltpu.async_copy(
                    y_hbm.at[idx_v.at[k0 + u + 2, 0]],
                    rows[(u + 2) % 3], gsem[(u + 2) % 3])
            dg[u].wait()
            ds[u] = pltpu.async_copy(
                rows[b], acc_sh.at[idx_v.at[k0 + u, 1]], ssem[b],
                add=True)
        for u in range(5, 8):
            ds[u].wait()
        return 0
    lax.fori_loop(0, nch // 8, step, 0)

    plsc.subcore_barrier()

    def out(p, _):
        r = row0 + 2 * p * CH
        pltpu.sync_copy(acc_sh.at[pl.ds(r, CH)], r0_v)
        d0 = pltpu.async_copy(r0_v, out_hbm.at[c, pl.ds(r, CH)], g0)
        pltpu.sync_copy(acc_sh.at[pl.ds(r + CH, CH)], r1_v)
        d1 = pltpu.async_copy(r1_v, out_hbm.at[c, pl.ds(r + CH, CH)], g1)
        d0.wait()
        d1.wait()
        return 0
    lax.fori_loop(0, RPT // CH // 2, out, 0)


def _k1_body(x_ref, w_ref, dp_ref, y_ref, dis_ref):
    i = pl.program_id(0)
    dp = dp_ref[...]
    dis = lax.rsqrt(dp[0] + dp[1] + 1.0)
    dis_ref[...] = dis
    y = dis[:, :1] * jnp.dot(x_ref[...], w_ref[...], preferred_element_type=f32)
    rows = i * BN + lax.broadcasted_iota(i32, (BN, 1), 0)
    y_ref[...] = jnp.where(rows < N, y, 0.0)


def _k1(xp, w1, degp):
    return pl.pallas_call(
        _k1_body,
        grid=(NB,),
        in_specs=[
            pl.BlockSpec((BN, D), lambda i: (i, 0)),
            pl.BlockSpec((D, D), lambda i: (0, 0)),
            pl.BlockSpec((NC, BN, 16), lambda i: (0, i, 0)),
        ],
        out_specs=[
            pl.BlockSpec((BN, D), lambda i: (i, 0)),
            pl.BlockSpec((BN, 16), lambda i: (i, 0)),
        ],
        out_shape=[
            jax.ShapeDtypeStruct((NP, D), f32),
            jax.ShapeDtypeStruct((NP, 16), f32),
        ],
    )(xp, w1, degp)


def _k23_body(sp_ref, y_ref, dis_ref, b_ref, w_ref, yn_ref):
    i = pl.program_id(0)
    sp = sp_ref[...]
    dis = dis_ref[...][:, :1]
    h = jnp.maximum(dis * (sp[0] + sp[1] + y_ref[...]) + b_ref[...], 0.0)
    yn = dis * jnp.dot(h, w_ref[...], preferred_element_type=f32)
    rows = i * BN + lax.broadcasted_iota(i32, (BN, 1), 0)
    yn_ref[...] = jnp.where(rows < N, yn, 0.0)


def _k23(sp, y, dis, b2d, w):
    return pl.pallas_call(
        _k23_body,
        grid=(NB,),
        in_specs=[
            pl.BlockSpec((NC, BN, D), lambda i: (0, i, 0)),
            pl.BlockSpec((BN, D), lambda i: (i, 0)),
            pl.BlockSpec((BN, 16), lambda i: (i, 0)),
            pl.BlockSpec((1, D), lambda i: (0, 0)),
            pl.BlockSpec((D, D), lambda i: (0, 0)),
        ],
        out_specs=pl.BlockSpec((BN, D), lambda i: (i, 0)),
        out_shape=jax.ShapeDtypeStruct((NP, D), f32),
    )(sp, y, dis, b2d, w)


def _k4_body(sp_ref, y_ref, dis_ref, b_ref, bat_ref, fw1_ref, fb1_ref,
             fw2_ref, fb2_ref, out_ref, acc_ref, cnt_ref):
    i = pl.program_id(0)

    @pl.when(i == 0)
    def _():
        acc_ref[...] = jnp.zeros((G, D), f32)
        cnt_ref[...] = jnp.zeros((G, 128), f32)

    sp = sp_ref[...]
    dis = dis_ref[...][:, :1]
    h = jnp.maximum(dis * (sp[0] + sp[1] + y_ref[...]) + b_ref[...], 0.0)
    onehot = (bat_ref[...][:, :1] == lax.broadcasted_iota(i32, (1, G), 1)
              ).astype(f32)
    acc_ref[...] += lax.dot_general(onehot, h, (((0,), (0,)), ((), ())),
                                    preferred_element_type=f32)
    cnt_ref[...] += jnp.broadcast_to(jnp.sum(onehot, axis=0)[:, None],
                                     (G, 128))

    @pl.when(i == NB - 1)
    def _():
        pooled = acc_ref[...] / jnp.maximum(cnt_ref[...], 1.0)
        z = jnp.maximum(jnp.dot(pooled, fw1_ref[...],
                                preferred_element_type=f32) + fb1_ref[...], 0.0)
        out_ref[...] = jnp.dot(z, fw2_ref[...],
                               preferred_element_type=f32) + fb2_ref[...]


def _k4(sp, y, dis, b2d, bat, fw1, fb1_2d, fw2, fb2_2d):
    return pl.pallas_call(
        _k4_body,
        grid=(NB,),
        in_specs=[
            pl.BlockSpec((NC, BN, D), lambda i: (0, i, 0)),
            pl.BlockSpec((BN, D), lambda i: (i, 0)),
            pl.BlockSpec((BN, 16), lambda i: (i, 0)),
            pl.BlockSpec((1, D), lambda i: (0, 0)),
            pl.BlockSpec((BN, 8), lambda i: (i, 0)),
            pl.BlockSpec((D, D), lambda i: (0, 0)),
            pl.BlockSpec((1, D), lambda i: (0, 0)),
            pl.BlockSpec((D, D), lambda i: (0, 0)),
            pl.BlockSpec((1, D), lambda i: (0, 0)),
        ],
        out_specs=pl.BlockSpec((G, D), lambda i: (0, 0)),
        out_shape=jax.ShapeDtypeStruct((G, D), f32),
        scratch_shapes=[pltpu.VMEM((G, D), f32), pltpu.VMEM((G, 128), f32)],
    )(sp, y, dis, b2d, bat, fw1, fb1_2d, fw2, fb2_2d)


def kernel(x, edge_index, batch, W1, b1, W2, b2, W3, b3, fW1, fb1, fW2, fb2):
    pad_e = EP - E
    srcp = jnp.concatenate([edge_index[0], jnp.full((pad_e,), N, i32)])
    dstp = jnp.concatenate([edge_index[1], jnp.full((pad_e,), N, i32)])
    ei_t = jnp.stack([srcp.reshape(TOTC, CH),
                      dstp.reshape(TOTC, CH)], axis=1)
    xp = jnp.zeros((NP, D), f32).at[:N, :].set(x)
    batp = jnp.broadcast_to(
        jnp.concatenate([batch, jnp.full((NP - N,), G, i32)])[:, None], (NP, 8))

    degp = _deg_kernel(ei_t)
    y1, dis = _k1(xp, W1, degp)
    s1 = _edge_kernel(y1, ei_t)
    y2 = _k23(s1, y1, dis, b1.reshape(1, D), W2)
    s2 = _edge_kernel(y2, ei_t)
    y3 = _k23(s2, y2, dis, b2.reshape(1, D), W3)
    s3 = _edge_kernel(y3, ei_t)
    out = _k4(s3, y3, dis, b3.reshape(1, D), batp,
              fW1, fb1.reshape(1, D), fW2, fb2.reshape(1, D))
    return out
